# baseline (speedup 1.0000x reference)
import functools

import jax
import jax.numpy as jnp
from jax import lax
from jax.experimental import pallas as pl
from jax.experimental.pallas import tpu as pltpu
from jax.experimental.pallas import tpu_sc as plsc

K = 8192
D = 32
B = 16384

BB = 128
RPG = 32
KB = 256
NKC = K // KB

NC = 2
NS = 16
NW = NC * NS
BPW = B // NW
GCH = 128
NCH = BPW // GCH


RW = 128
NKC2 = KB // RW


def _argmin_body(x_ref, cb_ref, cbb_ref, idx_ref, e2_ref):
    @pl.when(pl.program_id(0) == 0)
    def _():
        cbf = cb_ref[...]
        e2 = jnp.sum(cbf * cbf, axis=1)
        e2_ref[...] = jnp.broadcast_to(e2[None, :], (BB, K))

    lane = lax.broadcasted_iota(jnp.int32, (BB, RW), 1)

    def scan_half(h, x2b, xb2):
        bt = jnp.full((BB, RW), jnp.inf, jnp.float32)
        bc = jnp.zeros((BB, RW), jnp.int32)
        for kc in range(h * (NKC // 2), (h + 1) * (NKC // 2)):
            cbb = cbb_ref[pl.ds(kc * KB, KB), :]
            m2 = lax.dot_general(xb2, cbb, (((1,), (1,)), ((), ())),
                                 preferred_element_type=jnp.float32)
            e2c = e2_ref[:, pl.ds(kc * KB, KB)]
            for s in range(NKC2):
                c = kc * NKC2 + s
                ts = (x2b - m2[:, s * RW:(s + 1) * RW]) \
                    + e2c[:, s * RW:(s + 1) * RW]
                upd = ts < bt
                bc = jnp.where(upd, c, bc)
                bt = jnp.minimum(ts, bt)
        gm = jnp.min(bt, axis=1, keepdims=True)
        jc = jnp.where(bt == gm, bc * RW + lane, K)
        gi = jnp.min(jc, axis=1)
        return -gm[:, 0], gi

    for rb in range(RPG):
        x = x_ref[pl.ds(rb * BB, BB), :]
        x2 = jnp.sum(x * x, axis=1, keepdims=True)
        x2b = jnp.broadcast_to(x2, (BB, RW))
        xb2 = x.astype(jnp.bfloat16) * jnp.bfloat16(2.0)
        dA, iA = scan_half(0, x2b, xb2)
        dB, iB = scan_half(1, x2b, xb2)
        dAr = dA.astype(jnp.bfloat16).astype(jnp.float32)
        idx_ref[0, rb, :] = jnp.where(dB > dAr, iB, iA)


def _nearest_idx(x, codebook):
    return pl.pallas_call(
        _argmin_body,
        grid=(B // (RPG * BB),),
        in_specs=[
            pl.BlockSpec((RPG * BB, D), lambda i: (i, 0)),
            pl.BlockSpec((K, D), lambda i: (0, 0)),
            pl.BlockSpec((K, D), lambda i: (0, 0)),
        ],
        out_specs=pl.BlockSpec((1, RPG, BB), lambda i: (i, 0, 0)),
        out_shape=jax.ShapeDtypeStruct((B // (RPG * BB), RPG, BB), jnp.int32),
        scratch_shapes=[pltpu.VMEM((BB, K), jnp.float32)],
    )(x, codebook, codebook.astype(jnp.bfloat16)).reshape(B)


def _gather_rows(codebook, idx3):
    mesh = plsc.VectorSubcoreMesh(core_axis_name="c", subcore_axis_name="s")

    @functools.partial(
        pl.kernel,
        mesh=mesh,
        out_type=jax.ShapeDtypeStruct((B, D), jnp.float32),
        scratch_types=[
            pltpu.VMEM((NCH, GCH), jnp.int32),
            pltpu.VMEM((BPW, D), jnp.float32),
            pltpu.SemaphoreType.DMA,
        ],
        compiler_params=pltpu.CompilerParams(use_tc_tiling_on_sc=False),
    )
    def k(table_hbm, idx_hbm, out_hbm, idx_v, rows_v, sem):
        wid = lax.axis_index("s") * NC + lax.axis_index("c")
        pltpu.sync_copy(idx_hbm.at[wid], idx_v)
        for j in range(NCH):
            pltpu.async_copy(
                table_hbm.at[idx_v.at[j]],
                rows_v.at[pl.ds(j * GCH, GCH)],
                sem,
            ).wait()
        pltpu.sync_copy(rows_v, out_hbm.at[pl.ds(wid * BPW, BPW)])

    return k(codebook, idx3)


def kernel(x, codebook):
    idx = _nearest_idx(x, codebook)
    quantize = _gather_rows(codebook, idx.reshape(NW, NCH, GCH))
    return (quantize, idx, None)


# speedup vs baseline: 1.0021x; 1.0021x over previous
import functools

import jax
import jax.numpy as jnp
from jax import lax
from jax.experimental import pallas as pl
from jax.experimental.pallas import tpu as pltpu
from jax.experimental.pallas import tpu_sc as plsc

K = 8192
D = 32
B = 16384

BB = 128
RPG = 32
KB = 256
NKC = K // KB

NC = 2
NS = 16
NW = NC * NS
BPW = B // NW
GCH = 128
NCH = BPW // GCH


RW = 128
NKC2 = KB // RW


def _argmin_body(x_ref, cb_ref, cbb_ref, idx_ref, e2_ref):
    @pl.when(pl.program_id(0) == 0)
    def _():
        cbf = cb_ref[...]
        e2 = jnp.sum(cbf * cbf, axis=1)
        e2_ref[...] = jnp.broadcast_to(e2[None, :], (BB, K))

    lane = lax.broadcasted_iota(jnp.int32, (BB, RW), 1)

    def scan_half(h, x2b, xb2):
        bt = jnp.full((BB, RW), jnp.inf, jnp.float32)
        bc = jnp.zeros((BB, RW), jnp.int32)
        for kc in range(h * (NKC // 2), (h + 1) * (NKC // 2)):
            cbb = cbb_ref[pl.ds(kc * KB, KB), :]
            m2 = lax.dot_general(xb2, cbb, (((1,), (1,)), ((), ())),
                                 preferred_element_type=jnp.float32)
            for s in range(NKC2):
                c = kc * NKC2 + s
                e2s = e2_ref[:, pl.ds(c * RW, RW)]
                ts = (x2b - m2[:, s * RW:(s + 1) * RW]) + e2s
                upd = ts < bt
                bc = jnp.where(upd, c, bc)
                bt = jnp.minimum(ts, bt)
        gm = jnp.min(bt, axis=1, keepdims=True)
        jc = jnp.where(bt == gm, bc * RW + lane, K)
        gi = jnp.min(jc, axis=1)
        return -gm[:, 0], gi

    for rb in range(RPG):
        x = x_ref[pl.ds(rb * BB, BB), :]
        x2 = jnp.sum(x * x, axis=1, keepdims=True)
        x2b = jnp.broadcast_to(x2, (BB, RW))
        xb2 = x.astype(jnp.bfloat16) * jnp.bfloat16(2.0)
        dA, iA = scan_half(0, x2b, xb2)
        dB, iB = scan_half(1, x2b, xb2)
        dAr = dA.astype(jnp.bfloat16).astype(jnp.float32)
        idx_ref[0, rb, :] = jnp.where(dB > dAr, iB, iA)


def _nearest_idx(x, codebook):
    return pl.pallas_call(
        _argmin_body,
        grid=(B // (RPG * BB),),
        in_specs=[
            pl.BlockSpec((RPG * BB, D), lambda i: (i, 0)),
            pl.BlockSpec((K, D), lambda i: (0, 0)),
            pl.BlockSpec((K, D), lambda i: (0, 0)),
        ],
        out_specs=pl.BlockSpec((1, RPG, BB), lambda i: (i, 0, 0)),
        out_shape=jax.ShapeDtypeStruct((B // (RPG * BB), RPG, BB), jnp.int32),
        scratch_shapes=[pltpu.VMEM((BB, K), jnp.float32)],
    )(x, codebook, codebook.astype(jnp.bfloat16)).reshape(B)


def _gather_rows(codebook, idx3):
    mesh = plsc.VectorSubcoreMesh(core_axis_name="c", subcore_axis_name="s")

    @functools.partial(
        pl.kernel,
        mesh=mesh,
        out_type=jax.ShapeDtypeStruct((B, D), jnp.float32),
        scratch_types=[
            pltpu.VMEM((NCH, GCH), jnp.int32),
            pltpu.VMEM((BPW, D), jnp.float32),
            pltpu.SemaphoreType.DMA,
        ],
        compiler_params=pltpu.CompilerParams(use_tc_tiling_on_sc=False),
    )
    def k(table_hbm, idx_hbm, out_hbm, idx_v, rows_v, sem):
        wid = lax.axis_index("s") * NC + lax.axis_index("c")
        pltpu.sync_copy(idx_hbm.at[wid], idx_v)
        for j in range(NCH):
            pltpu.async_copy(
                table_hbm.at[idx_v.at[j]],
                rows_v.at[pl.ds(j * GCH, GCH)],
                sem,
            ).wait()
        pltpu.sync_copy(rows_v, out_hbm.at[pl.ds(wid * BPW, BPW)])

    return k(codebook, idx3)


def kernel(x, codebook):
    idx = _nearest_idx(x, codebook)
    quantize = _gather_rows(codebook, idx.reshape(NW, NCH, GCH))
    return (quantize, idx, None)
